# baseline (speedup 1.0000x reference)
import functools

import jax
import jax.numpy as jnp
from jax import lax
from jax.experimental import pallas as pl
from jax.experimental.pallas import tpu as pltpu
from jax.experimental.pallas import tpu_sc as plsc


def _xmom_body(x_ref, s_ref, m_ref):
    i = pl.program_id(0)
    x = x_ref[...]
    s = lax.dot_general(x, x, (((0,), (0,)), ((), ())),
                        preferred_element_type=jnp.float32)
    sm = jnp.sum(x, axis=0, keepdims=True)

    @pl.when(i == 0)
    def _():
        s_ref[...] = jnp.zeros_like(s_ref)
        m_ref[...] = jnp.zeros_like(m_ref)

    s_ref[...] += s
    m_ref[...] += sm


def _xmom(x, row_block):
    m, d = x.shape
    xp = x.reshape(m // 8, 8 * d)
    grid = (xp.shape[0] // row_block,)
    s88, sm = pl.pallas_call(
        _xmom_body,
        grid=grid,
        in_specs=[pl.BlockSpec((row_block, 8 * d), lambda i: (i, 0))],
        out_specs=[
            pl.BlockSpec((8 * d, 8 * d), lambda i: (0, 0)),
            pl.BlockSpec((1, 8 * d), lambda i: (0, 0)),
        ],
        out_shape=[
            jax.ShapeDtypeStruct((8 * d, 8 * d), jnp.float32),
            jax.ShapeDtypeStruct((1, 8 * d), jnp.float32),
        ],
    )(xp)
    s3 = s88.reshape(8, d, 8, d)
    s10 = jnp.einsum("tcte->ce", s3)
    m10 = sm.reshape(8, d).sum(axis=0)
    saug = jnp.zeros((d + 1, d + 1), jnp.float32)
    saug = saug.at[:d, :d].set(s10).at[d, :d].set(m10).at[d, d].set(float(m))
    return saug


def _make_sc_gather(rows_total, cin, chunk):
    info = plsc.get_sparse_core_info()
    nw = info.num_cores * info.num_subcores
    per_w = rows_total // nw
    n_chunks = per_w // chunk
    mesh = plsc.VectorSubcoreMesh(core_axis_name="c", subcore_axis_name="s")

    @functools.partial(
        pl.kernel,
        mesh=mesh,
        compiler_params=pltpu.CompilerParams(use_tc_tiling_on_sc=False),
        out_type=jax.ShapeDtypeStruct((rows_total, cin), jnp.float32),
        scratch_types=[
            pltpu.VMEM((chunk,), jnp.int32),
            pltpu.VMEM((chunk, cin), jnp.float32),
            pltpu.SemaphoreType.DMA,
        ],
    )
    def gather_k(table_hbm, idx_hbm, out_hbm, idx_v, rows_v, sem):
        wid = lax.axis_index("s") * info.num_cores + lax.axis_index("c")
        base = wid * per_w

        def body(i, carry):
            off = base + i * chunk
            pltpu.sync_copy(idx_hbm.at[pl.ds(off, chunk)], idx_v)
            pltpu.async_copy(table_hbm.at[idx_v], rows_v, sem).wait()
            pltpu.sync_copy(rows_v, out_hbm.at[pl.ds(off, chunk)])
            return carry

        lax.fori_loop(0, n_chunks, body, 0)

    return gather_k


def _main_body(pb, k, fg_ref, x_ref, ft_ref, wx_ref, p_ref, b2_ref,
               lm_ref, wc_ref, fp_ref, sfp_ref, sft_ref):
    step = pl.program_id(0)
    c2 = wc_ref.shape[0]
    c = c2 // 2
    y = jnp.dot(x_ref[...], wx_ref[...],
                preferred_element_type=jnp.float32)
    y = y + jnp.dot(fg_ref[...], p_ref[...],
                    preferred_element_type=jnp.float32)
    y = y + b2_ref[...]
    f2 = jnp.maximum(y, lm_ref[...] * y)
    logits = jnp.dot(f2, wc_ref[...],
                     preferred_element_type=jnp.float32)
    e2 = jnp.exp(logits)
    g2 = e2 * f2

    def red(a):
        return jnp.sum(a.reshape(pb, k // 2, c2), axis=1)

    re2 = red(e2)
    rg2 = red(g2)
    se = re2[:, :c] + re2[:, c:]
    num = rg2[:, :c] + rg2[:, c:]
    fp = num / se
    fp_ref[...] = fp

    ones = jnp.ones((pb, 1), jnp.float32)
    fpa = jnp.concatenate([fp, ones], axis=1)
    sfp = lax.dot_general(fpa, fpa, (((0,), (0,)), ((), ())),
                          preferred_element_type=jnp.float32)
    fta = jnp.concatenate([ft_ref[...], ones], axis=1)
    sft = lax.dot_general(fta, fta, (((0,), (0,)), ((), ())),
                          preferred_element_type=jnp.float32)

    @pl.when(step == 0)
    def _():
        sfp_ref[...] = jnp.zeros_like(sfp_ref)
        sft_ref[...] = jnp.zeros_like(sft_ref)

    sfp_ref[...] += sfp
    sft_ref[...] += sft


def _main(fg, x, feat2d, w1e, c1, wattn, pb):
    bn, cin = feat2d.shape
    k = fg.shape[0] // bn
    crel = w1e.shape[1]
    c = cin + crel
    c2 = 2 * c
    drel = x.shape[1]
    wx2 = jnp.zeros((2 * drel, c2), jnp.float32)
    wx2 = wx2.at[:drel, cin:c].set(w1e).at[drel:, c + cin:].set(w1e)
    eye = jnp.eye(cin, dtype=jnp.float32)
    p2 = jnp.zeros((2 * cin, c2), jnp.float32)
    p2 = p2.at[:cin, :cin].set(eye).at[cin:, c:c + cin].set(eye)
    b2 = jnp.zeros((c2,), jnp.float32)
    b2 = b2.at[cin:c].set(c1).at[c + cin:].set(c1)
    lm = jnp.ones((c2,), jnp.float32)
    lm = lm.at[cin:c].set(0.2).at[c + cin:].set(0.2)
    wc2 = jnp.zeros((c2, c2), jnp.float32)
    wc2 = wc2.at[:c, :c].set(wattn).at[c:, c:].set(wattn)

    fg2 = fg.reshape(bn * k // 2, 2 * cin)
    x2 = x.reshape(bn * k // 2, 2 * drel)
    rows = pb * k // 2
    grid = (bn // pb,)
    return pl.pallas_call(
        functools.partial(_main_body, pb, k),
        grid=grid,
        in_specs=[
            pl.BlockSpec((rows, 2 * cin), lambda i: (i, 0)),
            pl.BlockSpec((rows, 2 * drel), lambda i: (i, 0)),
            pl.BlockSpec((pb, cin), lambda i: (i, 0)),
            pl.BlockSpec(wx2.shape, lambda i: (0, 0)),
            pl.BlockSpec(p2.shape, lambda i: (0, 0)),
            pl.BlockSpec((1, c2), lambda i: (0, 0)),
            pl.BlockSpec((1, c2), lambda i: (0, 0)),
            pl.BlockSpec(wc2.shape, lambda i: (0, 0)),
        ],
        out_specs=[
            pl.BlockSpec((pb, c), lambda i: (i, 0)),
            pl.BlockSpec((c + 1, c + 1), lambda i: (0, 0)),
            pl.BlockSpec((cin + 1, cin + 1), lambda i: (0, 0)),
        ],
        out_shape=[
            jax.ShapeDtypeStruct((bn, c), jnp.float32),
            jax.ShapeDtypeStruct((c + 1, c + 1), jnp.float32),
            jax.ShapeDtypeStruct((cin + 1, cin + 1), jnp.float32),
        ],
    )(fg2, x2, feat2d, wx2, p2, b2[None, :], lm[None, :], wc2)


def _finish_body(ft_ref, fp_ref, wsc_ref, wo_ref, cc_ref, o_ref):
    y = jnp.dot(ft_ref[...], wsc_ref[...],
                preferred_element_type=jnp.float32)
    y = y + jnp.dot(fp_ref[...], wo_ref[...],
                    preferred_element_type=jnp.float32)
    y = y + cc_ref[...]
    o_ref[...] = jnp.maximum(y, 0.2 * y)


def _finish(feat2d, fp, wsce, woe, cc, fb):
    bn, cin = feat2d.shape
    c = fp.shape[1]
    cout = wsce.shape[1]
    grid = (bn // fb,)
    return pl.pallas_call(
        _finish_body,
        grid=grid,
        in_specs=[
            pl.BlockSpec((fb, cin), lambda i: (i, 0)),
            pl.BlockSpec((fb, c), lambda i: (i, 0)),
            pl.BlockSpec(wsce.shape, lambda i: (0, 0)),
            pl.BlockSpec(woe.shape, lambda i: (0, 0)),
            pl.BlockSpec(cc.shape, lambda i: (0, 0)),
        ],
        out_specs=pl.BlockSpec((fb, cout), lambda i: (i, 0)),
        out_shape=jax.ShapeDtypeStruct((bn, cout), jnp.float32),
    )(feat2d, fp, wsce, woe, cc)


def _fold_bn(saug, w, b, g, be, eps=1e-5):
    d = w.shape[0]
    m = saug[d, d]
    mean_x = saug[d, :d] / m
    cov = saug[:d, :d] / m - jnp.outer(mean_x, mean_x)
    mean_y = mean_x @ w + b
    var_y = jnp.sum(w * (cov @ w), axis=0)
    a = g / jnp.sqrt(var_y + eps)
    return w * a[None, :], (b - mean_y) * a + be


_sc_gather = None


def _get_sc_gather(rows_total, cin):
    global _sc_gather
    if _sc_gather is None:
        _sc_gather = _make_sc_gather(rows_total, cin, chunk=2048)
    return _sc_gather


def kernel(xyz, feature, ori_relative_feature, neighbors_idx, W1, b1, g1,
           be1, W_attn, W_out, b_out, g_out, be_out, W_sc, b_sc, g_sc,
           be_sc):
    b, n, k = neighbors_idx.shape
    cin = feature.shape[-1]
    drel = ori_relative_feature.shape[-1]

    x = ori_relative_feature.reshape(b * n * k, drel)
    saug = _xmom(x, row_block=4096)
    w1e, c1 = _fold_bn(saug, W1, b1, g1, be1)

    table = feature.reshape(b * n, cin)
    flat_idx = (neighbors_idx
                + (jnp.arange(b, dtype=jnp.int32) * n)[:, None, None])
    flat_idx = flat_idx.reshape(b * n * k)
    fg = _get_sc_gather(b * n * k, cin)(table, flat_idx)

    fp, sfp, sft = _main(fg, x, table, w1e, c1, W_attn, pb=512)

    wsce, csc = _fold_bn(sft, W_sc, b_sc, g_sc, be_sc)
    woe, co = _fold_bn(sfp, W_out, b_out, g_out, be_out)
    cc = (csc + co)[None, :]
    out = _finish(table, fp, wsce, woe, cc, fb=4096).reshape(b, n, -1)

    return (xyz, out, ori_relative_feature, neighbors_idx)


# speedup vs baseline: 1.1409x; 1.1409x over previous
import functools

import jax
import jax.numpy as jnp
from jax import lax
from jax.experimental import pallas as pl
from jax.experimental.pallas import tpu as pltpu
from jax.experimental.pallas import tpu_sc as plsc


def _xmom_body(x_ref, s_ref):
    i = pl.program_id(0)
    x = x_ref[...]
    ones = jnp.ones((x.shape[0], 1), x.dtype)
    xa = jnp.concatenate([x, ones], axis=1)
    s = lax.dot_general(xa, xa, (((0,), (0,)), ((), ())),
                        preferred_element_type=jnp.float32)

    @pl.when(i == 0)
    def _():
        s_ref[...] = jnp.zeros_like(s_ref)

    s_ref[...] += s


def _xmom(x, row_block):
    m, d = x.shape
    grid = (m // row_block,)
    return pl.pallas_call(
        _xmom_body,
        grid=grid,
        in_specs=[pl.BlockSpec((row_block, d), lambda i: (i, 0))],
        out_specs=pl.BlockSpec((d + 1, d + 1), lambda i: (0, 0)),
        out_shape=jax.ShapeDtypeStruct((d + 1, d + 1), jnp.float32),
    )(x)


def _make_sc_gather(rows_total, cin, chunk):
    info = plsc.get_sparse_core_info()
    nw = info.num_cores * info.num_subcores
    per_w = rows_total // nw
    n_chunks = per_w // chunk
    mesh = plsc.VectorSubcoreMesh(core_axis_name="c", subcore_axis_name="s")

    @functools.partial(
        pl.kernel,
        mesh=mesh,
        compiler_params=pltpu.CompilerParams(use_tc_tiling_on_sc=False),
        out_type=jax.ShapeDtypeStruct((rows_total, cin), jnp.float32),
        scratch_types=[
            pltpu.VMEM((chunk,), jnp.int32),
            pltpu.VMEM((chunk,), jnp.int32),
            pltpu.VMEM((chunk, cin), jnp.float32),
            pltpu.VMEM((chunk, cin), jnp.float32),
            pltpu.SemaphoreType.DMA,
            pltpu.SemaphoreType.DMA,
            pltpu.SemaphoreType.DMA,
            pltpu.SemaphoreType.DMA,
        ],
    )
    def gather_k(table_hbm, idx_hbm, out_hbm, idx0, idx1, rows0, rows1,
                 gsem0, gsem1, ssem0, ssem1):
        wid = lax.axis_index("s") * info.num_cores + lax.axis_index("c")
        base = wid * per_w
        idx_b = (idx0, idx1)
        rows_b = (rows0, rows1)
        gsem_b = (gsem0, gsem1)
        ssem_b = (ssem0, ssem1)

        def fire(c, s):
            pltpu.sync_copy(idx_hbm.at[pl.ds(base + c * chunk, chunk)],
                            idx_b[s])
            pltpu.async_copy(table_hbm.at[idx_b[s]], rows_b[s], gsem_b[s])

        fire(0, 0)
        fire(1, 1)

        def body(g0, carry):
            for s in (0, 1):
                c = 2 * g0 + s
                dst = out_hbm.at[pl.ds(base + c * chunk, chunk)]
                pltpu.make_async_copy(dst, rows_b[s], gsem_b[s]).wait()
                pltpu.async_copy(rows_b[s], dst, ssem_b[s])
                pltpu.make_async_copy(rows_b[s], dst, ssem_b[s]).wait()

                @pl.when(c + 2 < n_chunks)
                def _():
                    fire(c + 2, s)

            return carry

        lax.fori_loop(0, n_chunks // 2, body, 0)

    return gather_k


def _main_body(pb, k, fg_ref, x_ref, ft_ref, w1_ref, c1_ref, wa_ref,
               fp_ref, sfp_ref, sft_ref):
    step = pl.program_id(0)
    cin = fg_ref.shape[1]
    rel = jnp.dot(x_ref[...], w1_ref[...],
                  preferred_element_type=jnp.float32)
    rel = rel + c1_ref[...]
    rel = jnp.maximum(rel, 0.2 * rel)
    fg = fg_ref[...]
    wa = wa_ref[...]
    logits = jnp.dot(fg, wa[:cin, :],
                     preferred_element_type=jnp.float32)
    logits = logits + jnp.dot(rel, wa[cin:, :],
                              preferred_element_type=jnp.float32)
    def ksum(m):
        m3 = m.reshape(pb, k, m.shape[1])
        kk = k
        while kk > 1:
            kk //= 2
            m3 = m3[:, :kk, :] + m3[:, kk:, :]
        return m3[:, 0, :]

    e = jnp.exp(logits)
    se = ksum(e)
    nfg = ksum(e[:, :cin] * fg)
    nrel = ksum(e[:, cin:] * rel)
    fp = jnp.concatenate([nfg, nrel], axis=1) / se
    fp_ref[...] = fp

    ones = jnp.ones((pb, 1), jnp.float32)
    fpa = jnp.concatenate([fp, ones], axis=1)
    sfp = lax.dot_general(fpa, fpa, (((0,), (0,)), ((), ())),
                          preferred_element_type=jnp.float32)
    fta = jnp.concatenate([ft_ref[...], ones], axis=1)
    sft = lax.dot_general(fta, fta, (((0,), (0,)), ((), ())),
                          preferred_element_type=jnp.float32)

    @pl.when(step == 0)
    def _():
        sfp_ref[...] = jnp.zeros_like(sfp_ref)
        sft_ref[...] = jnp.zeros_like(sft_ref)

    sfp_ref[...] += sfp
    sft_ref[...] += sft


def _main(fg, x, feat2d, w1e, c1, wattn, pb):
    bn, cin = feat2d.shape
    k = fg.shape[0] // bn
    c = cin + w1e.shape[1]
    grid = (bn // pb,)
    return pl.pallas_call(
        functools.partial(_main_body, pb, k),
        grid=grid,
        in_specs=[
            pl.BlockSpec((pb * k, cin), lambda i: (i, 0)),
            pl.BlockSpec((pb * k, x.shape[1]), lambda i: (i, 0)),
            pl.BlockSpec((pb, cin), lambda i: (i, 0)),
            pl.BlockSpec(w1e.shape, lambda i: (0, 0)),
            pl.BlockSpec((1, c - cin), lambda i: (0, 0)),
            pl.BlockSpec(wattn.shape, lambda i: (0, 0)),
        ],
        out_specs=[
            pl.BlockSpec((pb, c), lambda i: (i, 0)),
            pl.BlockSpec((c + 1, c + 1), lambda i: (0, 0)),
            pl.BlockSpec((cin + 1, cin + 1), lambda i: (0, 0)),
        ],
        out_shape=[
            jax.ShapeDtypeStruct((bn, c), jnp.float32),
            jax.ShapeDtypeStruct((c + 1, c + 1), jnp.float32),
            jax.ShapeDtypeStruct((cin + 1, cin + 1), jnp.float32),
        ],
    )(fg, x, feat2d, w1e, c1[None, :], wattn)


def _finish_body(ft_ref, fp_ref, wsc_ref, wo_ref, cc_ref, o_ref):
    y = jnp.dot(ft_ref[...], wsc_ref[...],
                preferred_element_type=jnp.float32)
    y = y + jnp.dot(fp_ref[...], wo_ref[...],
                    preferred_element_type=jnp.float32)
    y = y + cc_ref[...]
    o_ref[...] = jnp.maximum(y, 0.2 * y)


def _finish(feat2d, fp, wsce, woe, cc, fb):
    bn, cin = feat2d.shape
    c = fp.shape[1]
    cout = wsce.shape[1]
    grid = (bn // fb,)
    return pl.pallas_call(
        _finish_body,
        grid=grid,
        in_specs=[
            pl.BlockSpec((fb, cin), lambda i: (i, 0)),
            pl.BlockSpec((fb, c), lambda i: (i, 0)),
            pl.BlockSpec(wsce.shape, lambda i: (0, 0)),
            pl.BlockSpec(woe.shape, lambda i: (0, 0)),
            pl.BlockSpec(cc.shape, lambda i: (0, 0)),
        ],
        out_specs=pl.BlockSpec((fb, cout), lambda i: (i, 0)),
        out_shape=jax.ShapeDtypeStruct((bn, cout), jnp.float32),
    )(feat2d, fp, wsce, woe, cc)


def _fold_bn(saug, w, b, g, be, eps=1e-5):
    d = w.shape[0]
    m = saug[d, d]
    mean_x = saug[d, :d] / m
    cov = saug[:d, :d] / m - jnp.outer(mean_x, mean_x)
    mean_y = mean_x @ w + b
    var_y = jnp.sum(w * (cov @ w), axis=0)
    a = g / jnp.sqrt(var_y + eps)
    return w * a[None, :], (b - mean_y) * a + be


_sc_gather = None


def _get_sc_gather(rows_total, cin):
    global _sc_gather
    if _sc_gather is None:
        _sc_gather = _make_sc_gather(rows_total, cin, chunk=1024)
    return _sc_gather


def kernel(xyz, feature, ori_relative_feature, neighbors_idx, W1, b1, g1,
           be1, W_attn, W_out, b_out, g_out, be_out, W_sc, b_sc, g_sc,
           be_sc):
    b, n, k = neighbors_idx.shape
    cin = feature.shape[-1]
    drel = ori_relative_feature.shape[-1]

    x = ori_relative_feature.reshape(b * n * k, drel)
    saug = _xmom(x, row_block=8192)
    w1e, c1 = _fold_bn(saug, W1, b1, g1, be1)

    table = feature.reshape(b * n, cin)
    flat_idx = (neighbors_idx
                + (jnp.arange(b, dtype=jnp.int32) * n)[:, None, None])
    flat_idx = flat_idx.reshape(b * n * k)
    fg = _get_sc_gather(b * n * k, cin)(table, flat_idx)

    fp, sfp, sft = _main(fg, x, table, w1e, c1, W_attn, pb=512)

    wsce, csc = _fold_bn(sft, W_sc, b_sc, g_sc, be_sc)
    woe, co = _fold_bn(sfp, W_out, b_out, g_out, be_out)
    cc = (csc + co)[None, :]
    out = _finish(table, fp, wsce, woe, cc, fb=4096).reshape(b, n, -1)

    return (xyz, out, ori_relative_feature, neighbors_idx)
